# baseline (speedup 1.0000x reference)
import functools

import jax
import jax.numpy as jnp
from jax import lax
from jax.experimental import pallas as pl
from jax.experimental.pallas import tpu as pltpu
from jax.experimental.pallas import tpu_sc as plsc

N = 100000
D = 128
C = 128
NP = 100096
NUM_CHUNKS = NP // C
TAIL_BASE = (NUM_CHUNKS - 1) * C
TAIL_ROWS = N - TAIL_BASE
NC, NS, L = 2, 16, 16
NW = NC * NS
NPAIRS = ((NUM_CHUNKS + NW - 1) // NW + 1) // 2
F = 7
G = 3

V1 = 22 * 17 * 14
V1_CHUNK = 320


def _body(idx7, t0, t1, t2,
          out_hbm, idx_v, fidx_v, rows_a, rows_b, t1_sh,
          sem_idx0, sem_idx1, sem_g00, sem_g01, sem_g10, sem_g11,
          sem_g20, sem_g21, sem_out0, sem_out1):
    sem_idx = (sem_idx0, sem_idx1)
    sem_g0 = (sem_g00, sem_g01)
    sem_g1 = (sem_g10, sem_g11)
    sem_g2 = (sem_g20, sem_g21)
    sem_out = (sem_out0, sem_out1)
    sid = lax.axis_index("s")
    wid = sid * NC + lax.axis_index("c")

    pltpu.sync_copy(t1.at[pl.ds(sid * V1_CHUNK, V1_CHUNK)],
                    t1_sh.at[pl.ds(sid * V1_CHUNK, V1_CHUNK)])

    @pl.when(sid == NS - 1)
    def _():
        pltpu.sync_copy(t1.at[pl.ds(NS * V1_CHUNK, V1 - NS * V1_CHUNK)],
                        t1_sh.at[pl.ds(NS * V1_CHUNK, V1 - NS * V1_CHUNK)])

    plsc.subcore_barrier()

    def base_of(i, b):
        c = wid + (2 * i + b) * NW
        return pl.multiple_of(c * C, C), c < NUM_CHUNKS

    def pair_body(i, _):
        bv = [base_of(i, b) for b in range(2)]
        bases = [bv[0][0], bv[1][0]]
        valids = [bv[0][1], bv[1][1]]

        for b in range(2):
            @pl.when(jnp.logical_and(valids[b], i > 0))
            def _():
                pltpu.make_async_copy(
                    rows_a.at[b], out_hbm.at[pl.ds(bases[b], C)],
                    sem_out[b]).wait()

            @pl.when(valids[b])
            def _():
                pltpu.async_copy(idx7.at[:, pl.ds(bases[b], C)],
                                 idx_v.at[b], sem_idx[b])

        for b in range(2):
            @pl.when(valids[b])
            def _():
                pltpu.make_async_copy(idx7.at[:, pl.ds(bases[b], C)],
                                      idx_v.at[b], sem_idx[b]).wait()
                for s in range(C // L):
                    sl = pl.ds(s * L, L)
                    fidx_v[b, 0, sl] = idx_v[b, 0, sl]
                    fidx_v[b, 1, sl] = ((idx_v[b, 1, sl] * 17
                                         + idx_v[b, 2, sl]) * 14
                                        + idx_v[b, 3, sl])
                    fidx_v[b, 2, sl] = ((idx_v[b, 4, sl] * 7
                                         + idx_v[b, 5, sl]) * 14
                                        + idx_v[b, 6, sl])
                pltpu.async_copy(t0.at[fidx_v.at[b, 0]], rows_a.at[b],
                                 sem_g0[b])
                pltpu.async_copy(t1_sh.at[fidx_v.at[b, 1]], rows_b.at[b],
                                 sem_g1[b])

        for b in range(2):
            @pl.when(valids[b])
            def _():
                pltpu.make_async_copy(t0.at[fidx_v.at[b, 0]], rows_a.at[b],
                                      sem_g0[b]).wait()
                pltpu.async_copy(t2.at[fidx_v.at[b, 2]], rows_a.at[b],
                                 sem_g2[b], add=True)

        for b in range(2):
            @pl.when(valids[b])
            def _():
                pltpu.make_async_copy(t1_sh.at[fidx_v.at[b, 1]], rows_b.at[b],
                                      sem_g1[b]).wait()
                pltpu.make_async_copy(t2.at[fidx_v.at[b, 2]], rows_a.at[b],
                                      sem_g2[b]).wait()

                def sum_body(r2, _):
                    for u in range(2):
                        r = r2 * 2 + u
                        for s in range(D // L):
                            sl = pl.ds(s * L, L)
                            plsc.addupdate(rows_a.at[b, r, sl],
                                           rows_b[b, r, sl])
                    return 0

                lax.fori_loop(0, C // 2, sum_body, 0)

                @pl.when(bases[b] != TAIL_BASE)
                def _():
                    pltpu.async_copy(rows_a.at[b],
                                     out_hbm.at[pl.ds(bases[b], C)],
                                     sem_out[b])

                @pl.when(bases[b] == TAIL_BASE)
                def _():
                    pltpu.async_copy(rows_a.at[b, pl.ds(0, TAIL_ROWS)],
                                     out_hbm.at[pl.ds(TAIL_BASE, TAIL_ROWS)],
                                     sem_out[b])
        return 0

    lax.fori_loop(0, NPAIRS, pair_body, 0)

    for b in range(2):
        base, valid = base_of(NPAIRS - 1, b)

        @pl.when(jnp.logical_and(valid, base != TAIL_BASE))
        def _():
            pltpu.make_async_copy(rows_a.at[b], out_hbm.at[pl.ds(base, C)],
                                  sem_out[b]).wait()

        @pl.when(jnp.logical_and(valid, base == TAIL_BASE))
        def _():
            pltpu.make_async_copy(rows_a.at[b, pl.ds(0, TAIL_ROWS)],
                                  out_hbm.at[pl.ds(TAIL_BASE, TAIL_ROWS)],
                                  sem_out[b]).wait()


@jax.jit
def kernel(atomic_num, formal_charge, degree, chiral_tag, total_numHs,
           is_aromatic, hybridization,
           W_atomic_num, W_formal_charge, W_degree, W_chiral_tag,
           W_total_numHs, W_is_aromatic, W_hybridization):
    t1 = (W_formal_charge[:, None, None, :] + W_degree[None, :, None, :]
          + W_chiral_tag[None, None, :, :]).reshape(-1, D)
    t2 = (W_total_numHs[:, None, None, :] + W_is_aromatic[None, :, None, :]
          + W_hybridization[None, None, :, :]).reshape(-1, D)
    idx7 = jnp.stack([atomic_num, formal_charge, degree, chiral_tag,
                      total_numHs, is_aromatic, hybridization])
    idx7 = jnp.pad(idx7, ((0, 0), (0, NP - N)))

    mesh = plsc.VectorSubcoreMesh(core_axis_name="c", subcore_axis_name="s")
    run = pl.kernel(
        _body,
        out_type=jax.ShapeDtypeStruct((N, D), jnp.float32),
        mesh=mesh,
        scratch_types=[
            pltpu.VMEM((2, F, C), jnp.int32),
            pltpu.VMEM((2, G, C), jnp.int32),
            pltpu.VMEM((2, C, D), jnp.float32),
            pltpu.VMEM((2, C, D), jnp.float32),
            pltpu.VMEM_SHARED((V1, D), jnp.float32),
        ] + [pltpu.SemaphoreType.DMA] * 10,
    )
    return run(idx7, W_atomic_num, t1, t2)


# speedup vs baseline: 1.1905x; 1.1905x over previous
import functools

import jax
import jax.numpy as jnp
from jax import lax
from jax.experimental import pallas as pl
from jax.experimental.pallas import tpu as pltpu
from jax.experimental.pallas import tpu_sc as plsc

N = 100000
D = 128
C = 80
NUM_CHUNKS = N // C
NC, NS, L = 2, 16, 16
NW = NC * NS
NBUF = 4
NGRP = ((NUM_CHUNKS + NW - 1) // NW + NBUF - 1) // NBUF
F = 7
G = 3

V1 = 22 * 17 * 14
V1_CHUNK = 320


def _body(a0, a1, a2, a3, a4, a5, a6,
          t0, t1, t2,
          out_hbm, idx_v, fidx_v, rows_a, rows_b, t1_sh,
          *sems):
    idx_hbm = (a0, a1, a2, a3, a4, a5, a6)
    sem_idx = sems[0:NBUF]
    sem_g0 = sems[NBUF:2 * NBUF]
    sem_g1 = sems[2 * NBUF:3 * NBUF]
    sem_g2 = sems[3 * NBUF:4 * NBUF]
    sem_out = sems[4 * NBUF:5 * NBUF]
    sid = lax.axis_index("s")
    wid = sid * NC + lax.axis_index("c")

    pltpu.sync_copy(t1.at[pl.ds(sid * V1_CHUNK, V1_CHUNK)],
                    t1_sh.at[pl.ds(sid * V1_CHUNK, V1_CHUNK)])

    @pl.when(sid == NS - 1)
    def _():
        pltpu.sync_copy(t1.at[pl.ds(NS * V1_CHUNK, V1 - NS * V1_CHUNK)],
                        t1_sh.at[pl.ds(NS * V1_CHUNK, V1 - NS * V1_CHUNK)])

    plsc.subcore_barrier()

    def chunk_of(i, b):
        return wid + (NBUF * i + b) * NW

    def group_body(i, _):
        chunks = [chunk_of(i, b) for b in range(NBUF)]
        valids = [c < NUM_CHUNKS for c in chunks]
        bases = [c * C for c in chunks]

        for b in range(NBUF):
            @pl.when(jnp.logical_and(valids[b], i > 0))
            def _():
                pltpu.make_async_copy(
                    rows_a.at[b], out_hbm.at[pl.ds(bases[b], C)],
                    sem_out[b]).wait()

            @pl.when(valids[b])
            def _():
                for f in range(F):
                    pltpu.async_copy(idx_hbm[f].at[pl.ds(bases[b], C)],
                                     idx_v.at[b, f], sem_idx[b])

        for b in range(NBUF):
            @pl.when(valids[b])
            def _():
                for f in range(F):
                    pltpu.make_async_copy(idx_hbm[f].at[pl.ds(bases[b], C)],
                                          idx_v.at[b, f], sem_idx[b]).wait()
                for s in range(C // L):
                    sl = pl.ds(s * L, L)
                    fidx_v[b, 0, sl] = idx_v[b, 0, sl]
                    fidx_v[b, 1, sl] = ((idx_v[b, 1, sl] * 17
                                         + idx_v[b, 2, sl]) * 14
                                        + idx_v[b, 3, sl])
                    fidx_v[b, 2, sl] = ((idx_v[b, 4, sl] * 7
                                         + idx_v[b, 5, sl]) * 14
                                        + idx_v[b, 6, sl])
                pltpu.async_copy(t0.at[fidx_v.at[b, 0]], rows_a.at[b],
                                 sem_g0[b])
                pltpu.async_copy(t1_sh.at[fidx_v.at[b, 1]], rows_b.at[b],
                                 sem_g1[b])

        for b in range(NBUF):
            @pl.when(valids[b])
            def _():
                pltpu.make_async_copy(t0.at[fidx_v.at[b, 0]], rows_a.at[b],
                                      sem_g0[b]).wait()
                pltpu.async_copy(t2.at[fidx_v.at[b, 2]], rows_a.at[b],
                                 sem_g2[b], add=True)

        for b in range(NBUF):
            @pl.when(valids[b])
            def _():
                pltpu.make_async_copy(t1_sh.at[fidx_v.at[b, 1]], rows_b.at[b],
                                      sem_g1[b]).wait()
                pltpu.make_async_copy(t2.at[fidx_v.at[b, 2]], rows_a.at[b],
                                      sem_g2[b]).wait()

                def sum_body(r2, _):
                    for u in range(2):
                        r = r2 * 2 + u
                        for s in range(D // L):
                            sl = pl.ds(s * L, L)
                            plsc.addupdate(rows_a.at[b, r, sl],
                                           rows_b[b, r, sl])
                    return 0

                lax.fori_loop(0, C // 2, sum_body, 0)
                pltpu.async_copy(rows_a.at[b], out_hbm.at[pl.ds(bases[b], C)],
                                 sem_out[b])
        return 0

    lax.fori_loop(0, NGRP, group_body, 0)

    for b in range(NBUF):
        c = chunk_of(NGRP - 1, b)

        @pl.when(c < NUM_CHUNKS)
        def _():
            pltpu.make_async_copy(rows_a.at[b], out_hbm.at[pl.ds(c * C, C)],
                                  sem_out[b]).wait()


@jax.jit
def kernel(atomic_num, formal_charge, degree, chiral_tag, total_numHs,
           is_aromatic, hybridization,
           W_atomic_num, W_formal_charge, W_degree, W_chiral_tag,
           W_total_numHs, W_is_aromatic, W_hybridization):
    t1 = (W_formal_charge[:, None, None, :] + W_degree[None, :, None, :]
          + W_chiral_tag[None, None, :, :]).reshape(-1, D)
    t2 = (W_total_numHs[:, None, None, :] + W_is_aromatic[None, :, None, :]
          + W_hybridization[None, None, :, :]).reshape(-1, D)

    mesh = plsc.VectorSubcoreMesh(core_axis_name="c", subcore_axis_name="s")
    run = pl.kernel(
        _body,
        out_type=jax.ShapeDtypeStruct((N, D), jnp.float32),
        mesh=mesh,
        scratch_types=[
            pltpu.VMEM((NBUF, F, C), jnp.int32),
            pltpu.VMEM((NBUF, G, C), jnp.int32),
            pltpu.VMEM((NBUF, C, D), jnp.float32),
            pltpu.VMEM((NBUF, C, D), jnp.float32),
            pltpu.VMEM_SHARED((V1, D), jnp.float32),
        ] + [pltpu.SemaphoreType.DMA] * (5 * NBUF),
    )
    return run(atomic_num, formal_charge, degree, chiral_tag, total_numHs,
               is_aromatic, hybridization, W_atomic_num, t1, t2)


# speedup vs baseline: 1.8464x; 1.5510x over previous
import functools

import jax
import jax.numpy as jnp
from jax import lax
from jax.experimental import pallas as pl
from jax.experimental.pallas import tpu as pltpu
from jax.experimental.pallas import tpu_sc as plsc

N = 100000
D = 128
C = 80
NUM_CHUNKS = N // C
NC, NS, L = 2, 16, 16
NW = NC * NS
NBUF = 4
NGRP = ((NUM_CHUNKS + NW - 1) // NW + NBUF - 1) // NBUF
F = 7
G = 3

V0 = 124
V1 = 22 * 17 * 14
V1_CHUNK = 320


def _body(a0, a1, a2, a3, a4, a5, a6,
          t0, t1, t2,
          out_hbm, idx_v, fidx_v, rows_a, rows_b, t0_sh, t1_sh,
          *sems):
    idx_hbm = (a0, a1, a2, a3, a4, a5, a6)
    sem_idx = sems[0:NBUF]
    sem_g0 = sems[NBUF:2 * NBUF]
    sem_g1 = sems[2 * NBUF:3 * NBUF]
    sem_g2 = sems[3 * NBUF:4 * NBUF]
    sem_out = sems[4 * NBUF:5 * NBUF]
    sid = lax.axis_index("s")
    wid = sid * NC + lax.axis_index("c")

    pltpu.sync_copy(t1.at[pl.ds(sid * V1_CHUNK, V1_CHUNK)],
                    t1_sh.at[pl.ds(sid * V1_CHUNK, V1_CHUNK)])

    @pl.when(sid == NS - 1)
    def _():
        pltpu.sync_copy(t1.at[pl.ds(NS * V1_CHUNK, V1 - NS * V1_CHUNK)],
                        t1_sh.at[pl.ds(NS * V1_CHUNK, V1 - NS * V1_CHUNK)])

    @pl.when(sid == NS - 2)
    def _():
        pltpu.sync_copy(t0, t0_sh)

    plsc.subcore_barrier()

    def chunk_of(i, b):
        return wid + (NBUF * i + b) * NW

    def group_body(i, _):
        chunks = [chunk_of(i, b) for b in range(NBUF)]
        valids = [c < NUM_CHUNKS for c in chunks]
        bases = [c * C for c in chunks]

        for b in range(NBUF):
            @pl.when(jnp.logical_and(valids[b], i > 0))
            def _():
                pltpu.make_async_copy(
                    rows_a.at[b], out_hbm.at[pl.ds(bases[b], C)],
                    sem_out[b]).wait()

            @pl.when(valids[b])
            def _():
                for f in range(F):
                    pltpu.async_copy(idx_hbm[f].at[pl.ds(bases[b], C)],
                                     idx_v.at[b, f], sem_idx[b])

        for b in range(NBUF):
            @pl.when(valids[b])
            def _():
                for f in range(F):
                    pltpu.make_async_copy(idx_hbm[f].at[pl.ds(bases[b], C)],
                                          idx_v.at[b, f], sem_idx[b]).wait()
                for s in range(C // L):
                    sl = pl.ds(s * L, L)
                    fidx_v[b, 0, sl] = idx_v[b, 0, sl]
                    fidx_v[b, 1, sl] = ((idx_v[b, 1, sl] * 17
                                         + idx_v[b, 2, sl]) * 14
                                        + idx_v[b, 3, sl])
                    fidx_v[b, 2, sl] = ((idx_v[b, 4, sl] * 7
                                         + idx_v[b, 5, sl]) * 14
                                        + idx_v[b, 6, sl])
                pltpu.async_copy(t0_sh.at[fidx_v.at[b, 0]], rows_a.at[b],
                                 sem_g0[b])
                pltpu.async_copy(t1_sh.at[fidx_v.at[b, 1]], rows_b.at[b],
                                 sem_g1[b])

        for b in range(NBUF):
            @pl.when(valids[b])
            def _():
                pltpu.make_async_copy(t0_sh.at[fidx_v.at[b, 0]], rows_a.at[b],
                                      sem_g0[b]).wait()
                pltpu.async_copy(t2.at[fidx_v.at[b, 2]], rows_a.at[b],
                                 sem_g2[b], add=True)

        for b in range(NBUF):
            @pl.when(valids[b])
            def _():
                pltpu.make_async_copy(t1_sh.at[fidx_v.at[b, 1]], rows_b.at[b],
                                      sem_g1[b]).wait()
                pltpu.make_async_copy(t2.at[fidx_v.at[b, 2]], rows_a.at[b],
                                      sem_g2[b]).wait()

                def sum_body(r2, _):
                    for u in range(2):
                        r = r2 * 2 + u
                        for s in range(D // L):
                            sl = pl.ds(s * L, L)
                            plsc.addupdate(rows_a.at[b, r, sl],
                                           rows_b[b, r, sl])
                    return 0

                lax.fori_loop(0, C // 2, sum_body, 0)
                pltpu.async_copy(rows_a.at[b], out_hbm.at[pl.ds(bases[b], C)],
                                 sem_out[b])
        return 0

    lax.fori_loop(0, NGRP, group_body, 0)

    for b in range(NBUF):
        c = chunk_of(NGRP - 1, b)

        @pl.when(c < NUM_CHUNKS)
        def _():
            pltpu.make_async_copy(rows_a.at[b], out_hbm.at[pl.ds(c * C, C)],
                                  sem_out[b]).wait()


@jax.jit
def kernel(atomic_num, formal_charge, degree, chiral_tag, total_numHs,
           is_aromatic, hybridization,
           W_atomic_num, W_formal_charge, W_degree, W_chiral_tag,
           W_total_numHs, W_is_aromatic, W_hybridization):
    t1 = (W_formal_charge[:, None, None, :] + W_degree[None, :, None, :]
          + W_chiral_tag[None, None, :, :]).reshape(-1, D)
    t2 = (W_total_numHs[:, None, None, :] + W_is_aromatic[None, :, None, :]
          + W_hybridization[None, None, :, :]).reshape(-1, D)

    mesh = plsc.VectorSubcoreMesh(core_axis_name="c", subcore_axis_name="s")
    run = pl.kernel(
        _body,
        out_type=jax.ShapeDtypeStruct((N, D), jnp.float32),
        mesh=mesh,
        scratch_types=[
            pltpu.VMEM((NBUF, F, C), jnp.int32),
            pltpu.VMEM((NBUF, G, C), jnp.int32),
            pltpu.VMEM((NBUF, C, D), jnp.float32),
            pltpu.VMEM((NBUF, C, D), jnp.float32),
            pltpu.VMEM_SHARED((V0, D), jnp.float32),
            pltpu.VMEM_SHARED((V1, D), jnp.float32),
        ] + [pltpu.SemaphoreType.DMA] * (5 * NBUF),
    )
    return run(atomic_num, formal_charge, degree, chiral_tag, total_numHs,
               is_aromatic, hybridization, W_atomic_num, t1, t2)
